# baseline (speedup 1.0000x reference)
import functools

import jax
import jax.numpy as jnp
from jax import lax
from jax.experimental import pallas as pl
from jax.experimental.pallas import tpu as pltpu
from jax.experimental.pallas import tpu_sc as plsc

RES = 256
N = 1048576
NW = 32
P = N // NW
C = 512
G = C // 16
NCHUNK = P // C
NROW8 = RES * RES * RES * 3 // 8
S4 = (4 * C).bit_length() - 4

_mesh = plsc.VectorSubcoreMesh(core_axis_name="c", subcore_axis_name="s")


@functools.partial(
    pl.kernel,
    mesh=_mesh,
    out_type=jax.ShapeDtypeStruct((N, 3), jnp.float32),
    compiler_params=pltpu.CompilerParams(
        use_tc_tiling_on_sc=False, needs_layout_passes=False),
    scratch_types=[
        pltpu.VMEM((C, 3), jnp.float32),
        pltpu.VMEM((8 * C,), jnp.int32),
        pltpu.VMEM((4 * C,), jnp.int32),
        pltpu.VMEM((8 * C, 8), jnp.float32),
        pltpu.VMEM((3, C), jnp.float32),
        pltpu.VMEM((C, 3), jnp.float32),
        pltpu.SemaphoreType.DMA,
    ],
)
def _trilerp(pts_hbm, tab8_hbm, out_hbm, pts_v, idx_v, o_v, rows_v, t_v,
             out_v, sem):
    wid = lax.axis_index("s") * 2 + lax.axis_index("c")
    lanes = lax.iota(jnp.int32, 16)

    def chunk_body(i, _):
        base = wid * P + i * C
        pltpu.sync_copy(pts_hbm.at[pl.ds(base, C)], pts_v)

        def idx_body(g, _):
            p = g * 16 + lanes
            lo = []
            for ch in range(3):
                coord = plsc.load_gather(
                    pts_v, [p, jnp.full((16,), ch, jnp.int32)])
                s = coord * jnp.float32(RES - 1)
                li = jnp.minimum(s.astype(jnp.int32), RES - 2)
                t_v[ch, pl.ds(g * 16, 16)] = s - li.astype(jnp.float32)
                lo.append(li)
            ix, iy, iz = lo
            flat = (iz * RES + iy) * RES + ix
            for m in range(4):
                dz, dy = (m >> 1) & 1, m & 1
                e = (flat + dz * RES * RES + dy * RES) * 3
                j0 = e >> 3
                o_v[pl.ds(m * C + g * 16, 16)] = e & 7
                idx_v[pl.ds(m * C + g * 16, 16)] = j0
                idx_v[pl.ds((4 + m) * C + g * 16, 16)] = jnp.minimum(
                    j0 + 1, NROW8 - 1)
            return 0

        lax.fori_loop(0, G, idx_body, 0)

        pltpu.async_copy(tab8_hbm.at[idx_v], rows_v, sem).wait()

        def comb_body(g, _):
            p = g * 16 + lanes
            tx = t_v[0, pl.ds(g * 16, 16)]
            ty = t_v[1, pl.ds(g * 16, 16)]
            tz = t_v[2, pl.ds(g * 16, 16)]
            one = jnp.float32(1.0)
            wx = (one - tx, tx)
            acc = [None, None, None]
            for m in range(4):
                dz, dy = (m >> 1) & 1, m & 1
                az = (one - tz) if dz == 0 else tz
                ay = (one - ty) if dy == 0 else ty
                wzy = az * ay
                o = o_v[pl.ds(m * C + g * 16, 16)]
                bm = m * C + p
                for dx in range(2):
                    w = wzy * wx[dx]
                    for ch in range(3):
                        off = o + (dx * 3 + ch)
                        row = bm + ((off & 8) << S4)
                        col = off & 7
                        v = plsc.load_gather(rows_v, [row, col])
                        acc[ch] = w * v if acc[ch] is None else acc[ch] + w * v
            for ch in range(3):
                plsc.store_scatter(
                    out_v, [p, jnp.full((16,), ch, jnp.int32)], acc[ch])
            return 0

        lax.fori_loop(0, G, comb_body, 0)

        pltpu.sync_copy(out_v, out_hbm.at[pl.ds(base, C)])
        return 0

    lax.fori_loop(0, NCHUNK, chunk_body, 0)


def kernel(input, feature_params):
    tab8 = feature_params.reshape(NROW8, 8)
    return _trilerp(input, tab8)


# speedup vs baseline: 6.9993x; 6.9993x over previous
import functools

import jax
import jax.numpy as jnp
from jax import lax
from jax.experimental import pallas as pl
from jax.experimental.pallas import tpu as pltpu
from jax.experimental.pallas import tpu_sc as plsc

RES = 256
N = 1048576
NW = 32
P = N // NW
C = 256
G = C // 16
NCHUNK = P // C
NROW8 = RES * RES * RES * 3 // 8

_mesh = plsc.VectorSubcoreMesh(core_axis_name="c", subcore_axis_name="s")


@functools.partial(
    pl.kernel,
    mesh=_mesh,
    out_type=jax.ShapeDtypeStruct((N, 3), jnp.float32),
    compiler_params=pltpu.CompilerParams(
        use_tc_tiling_on_sc=False, needs_layout_passes=False),
    scratch_types=[
        pltpu.VMEM((C, 3), jnp.float32),
        pltpu.VMEM((24 * C,), jnp.int32),
        pltpu.VMEM((2, C), jnp.int32),
        pltpu.VMEM((24 * C, 8), jnp.float32),
        pltpu.VMEM((3, C), jnp.float32),
        pltpu.VMEM((C, 3), jnp.float32),
        pltpu.SemaphoreType.DMA,
    ],
)
def _trilerp(pts_hbm, tab8_hbm, out_hbm, pts_v, idx_v, cl_v, rows_v, t_v,
             out_v, sem):
    wid = lax.axis_index("s") * 2 + lax.axis_index("c")
    lanes = lax.iota(jnp.int32, 16)

    def chunk_body(i, _):
        base = wid * P + i * C
        pltpu.sync_copy(pts_hbm.at[pl.ds(base, C)], pts_v)

        def idx_body(g, _):
            g16 = g * 16
            p = g16 + lanes
            lo = []
            for ch in range(3):
                coord = plsc.load_gather(
                    pts_v, [p, jnp.full((16,), ch, jnp.int32)])
                s = coord * jnp.float32(RES - 1)
                li = jnp.minimum(s.astype(jnp.int32), RES - 2)
                t_v[ch, pl.ds(g16, 16)] = s - li.astype(jnp.float32)
                lo.append(li)
            ix, iy, iz = lo
            izc = iz * 3
            iy1 = iy + 1
            ix1 = ix + 1
            yt = (((iy >> 3) << 8) + ((iy & 7) << 4),
                  ((iy1 >> 3) << 8) + ((iy1 & 7) << 4))
            xt_lo = ((ix >> 7) << 7) + ((ix >> 3) & 15)
            xt_hi = ((ix1 >> 7) << 7) + ((ix1 >> 3) & 15)
            cl_v[0, pl.ds(g16, 16)] = ix & 7
            cl_v[1, pl.ds(g16, 16)] = ix1 & 7
            yx = ((yt[0] + xt_lo, yt[0] + xt_hi),
                  (yt[1] + xt_lo, yt[1] + xt_hi))
            for dz in range(2):
                for ch in range(3):
                    zterm = (izc + (dz * 3 + ch)) << 13
                    for dy in range(2):
                        m12 = (dz * 3 + ch) * 2 + dy
                        idx_v[pl.ds(m12 * C + g16, 16)] = zterm + yx[dy][0]
                        idx_v[pl.ds((12 + m12) * C + g16, 16)] = (
                            zterm + yx[dy][1])
            return 0

        lax.fori_loop(0, G, idx_body, 0)

        pltpu.async_copy(tab8_hbm.at[idx_v], rows_v, sem).wait()

        def comb_body(g, _):
            g16 = g * 16
            p = g16 + lanes
            cl = cl_v[0, pl.ds(g16, 16)]
            ch_ = cl_v[1, pl.ds(g16, 16)]
            tx = t_v[0, pl.ds(g16, 16)]
            ty = t_v[1, pl.ds(g16, 16)]
            tz = t_v[2, pl.ds(g16, 16)]
            one = jnp.float32(1.0)
            wx0 = one - tx
            wy = (one - ty, ty)
            wz = (one - tz, tz)
            acc = [None, None, None]
            for dz in range(2):
                for dy in range(2):
                    wzy = wz[dz] * wy[dy]
                    for ch in range(3):
                        m12 = (dz * 3 + ch) * 2 + dy
                        v_lo = plsc.load_gather(rows_v, [m12 * C + p, cl])
                        v_hi = plsc.load_gather(
                            rows_v, [(12 + m12) * C + p, ch_])
                        xv = v_lo + tx * (v_hi - v_lo)
                        acc[ch] = (wzy * xv if acc[ch] is None
                                   else acc[ch] + wzy * xv)
            del wx0
            for ch in range(3):
                plsc.store_scatter(
                    out_v, [p, jnp.full((16,), ch, jnp.int32)], acc[ch])
            return 0

        lax.fori_loop(0, G, comb_body, 0)

        pltpu.sync_copy(out_v, out_hbm.at[pl.ds(base, C)])
        return 0

    lax.fori_loop(0, NCHUNK, chunk_body, 0)


def kernel(input, feature_params):
    tab8 = (feature_params.transpose(0, 3, 1, 2)
            .reshape(RES, 3, 32, 8, 2, 128)
            .transpose(0, 1, 2, 4, 3, 5)
            .reshape(NROW8, 8))
    return _trilerp(input, tab8)


# speedup vs baseline: 7.5216x; 1.0746x over previous
import functools

import jax
import jax.numpy as jnp
from jax import lax
from jax.experimental import pallas as pl
from jax.experimental.pallas import tpu as pltpu
from jax.experimental.pallas import tpu_sc as plsc

RES = 256
N = 1048576
NW = 32
P = N // NW
C = 512
G = C // 16
NCHUNK = P // C
CR = 3 * C // 8
NROW8 = RES * RES * RES * 3 // 8

_mesh = plsc.VectorSubcoreMesh(core_axis_name="c", subcore_axis_name="s")


@functools.partial(
    pl.kernel,
    mesh=_mesh,
    out_type=jax.ShapeDtypeStruct((3 * N // 8, 8), jnp.float32),
    compiler_params=pltpu.CompilerParams(
        use_tc_tiling_on_sc=False, needs_layout_passes=False),
    scratch_types=[
        pltpu.VMEM((CR, 8), jnp.float32),
        pltpu.VMEM((24 * C,), jnp.int32),
        pltpu.VMEM((2, C), jnp.int32),
        pltpu.VMEM((24 * C, 8), jnp.float32),
        pltpu.VMEM((3, C), jnp.float32),
        pltpu.VMEM((CR, 8), jnp.float32),
        pltpu.SemaphoreType.DMA,
    ],
)
def _trilerp(pts_hbm, tab8_hbm, out_hbm, pts_v, idx_v, cl_v, rows_v, t_v,
             out_v, sem):
    wid = lax.axis_index("s") * 2 + lax.axis_index("c")
    lanes = lax.iota(jnp.int32, 16)

    def chunk_body(i, _):
        base = wid * P + i * C
        pltpu.sync_copy(pts_hbm.at[pl.ds(base * 3 // 8, CR)], pts_v)

        def idx_body(g, _):
            g16 = g * 16
            p = g16 + lanes
            e0 = p * 3
            lo = []
            for ch in range(3):
                e = e0 + ch
                coord = plsc.load_gather(pts_v, [e >> 3, e & 7])
                s = coord * jnp.float32(RES - 1)
                li = jnp.minimum(s.astype(jnp.int32), RES - 2)
                t_v[ch, pl.ds(g16, 16)] = s - li.astype(jnp.float32)
                lo.append(li)
            ix, iy, iz = lo
            izc = iz * 3
            iy1 = iy + 1
            ix1 = ix + 1
            yt = (((iy >> 3) << 8) + ((iy & 7) << 4),
                  ((iy1 >> 3) << 8) + ((iy1 & 7) << 4))
            xt_lo = ((ix >> 7) << 7) + ((ix >> 3) & 15)
            xt_hi = ((ix1 >> 7) << 7) + ((ix1 >> 3) & 15)
            cl_v[0, pl.ds(g16, 16)] = ix & 7
            cl_v[1, pl.ds(g16, 16)] = ix1 & 7
            yx = ((yt[0] + xt_lo, yt[0] + xt_hi),
                  (yt[1] + xt_lo, yt[1] + xt_hi))
            for dz in range(2):
                for ch in range(3):
                    zterm = (izc + (dz * 3 + ch)) << 13
                    for dy in range(2):
                        m12 = (dz * 3 + ch) * 2 + dy
                        idx_v[pl.ds(m12 * C + g16, 16)] = zterm + yx[dy][0]
                        idx_v[pl.ds((12 + m12) * C + g16, 16)] = (
                            zterm + yx[dy][1])
            return 0

        lax.fori_loop(0, G, idx_body, 0)

        pltpu.async_copy(tab8_hbm.at[idx_v], rows_v, sem).wait()

        def comb_body(g, _):
            g16 = g * 16
            p = g16 + lanes
            cl = cl_v[0, pl.ds(g16, 16)]
            ch_ = cl_v[1, pl.ds(g16, 16)]
            tx = t_v[0, pl.ds(g16, 16)]
            ty = t_v[1, pl.ds(g16, 16)]
            tz = t_v[2, pl.ds(g16, 16)]
            one = jnp.float32(1.0)
            wy = (one - ty, ty)
            wz = (one - tz, tz)
            acc = [None, None, None]
            for dz in range(2):
                for dy in range(2):
                    wzy = wz[dz] * wy[dy]
                    for ch in range(3):
                        m12 = (dz * 3 + ch) * 2 + dy
                        v_lo = plsc.load_gather(rows_v, [m12 * C + p, cl])
                        v_hi = plsc.load_gather(
                            rows_v, [(12 + m12) * C + p, ch_])
                        xv = v_lo + tx * (v_hi - v_lo)
                        acc[ch] = (wzy * xv if acc[ch] is None
                                   else acc[ch] + wzy * xv)
            e0 = p * 3
            for ch in range(3):
                e = e0 + ch
                plsc.store_scatter(out_v, [e >> 3, e & 7], acc[ch])
            return 0

        lax.fori_loop(0, G, comb_body, 0)

        pltpu.sync_copy(out_v, out_hbm.at[pl.ds(base * 3 // 8, CR)])
        return 0

    lax.fori_loop(0, NCHUNK, chunk_body, 0)


def kernel(input, feature_params):
    tab8 = (feature_params.transpose(0, 3, 1, 2)
            .reshape(RES, 3, 32, 8, 2, 128)
            .transpose(0, 1, 2, 4, 3, 5)
            .reshape(NROW8, 8))
    pts8 = input.reshape(3 * N // 8, 8)
    out8 = _trilerp(pts8, tab8)
    return out8.reshape(N, 3)
